# Initial kernel scaffold; baseline (speedup 1.0000x reference)
import jax
import jax.numpy as jnp
from jax.experimental import pallas as pl


def kernel(latent):
    raise NotImplementedError("write your pallas kernel here")


# speedup vs baseline: 10.6452x; 10.6452x over previous
import functools

import jax
import jax.numpy as jnp
from jax.experimental import pallas as pl
from jax.experimental.pallas import tpu as pltpu

_BLOCK_ROWS = 512
_K = 5


def _knn_moments_kernel(rows_ref, full_ref, out_ref, *, t, k, n_blocks):
    i = pl.program_id(0)

    x = rows_ref[...]
    y = full_ref[...]
    sqx = jnp.sum(x * x, axis=1, keepdims=True)
    sqy = jnp.sum(y * y, axis=1, keepdims=True).T
    xy = jax.lax.dot_general(
        x.astype(jnp.bfloat16), y.astype(jnp.bfloat16),
        dimension_numbers=(((1,), (1,)), ((), ())),
        preferred_element_type=jnp.float32,
    )
    d2 = jnp.maximum(sqx + sqy - 2.0 * xy, 0.0)
    dist = jnp.where(d2 > 0, jnp.sqrt(jnp.where(d2 > 0, d2, 1.0)), 0.0)

    col = jax.lax.broadcasted_iota(jnp.int32, dist.shape, 1)
    s = jnp.float32(0.0)
    ss = jnp.float32(0.0)
    for _ in range(k):
        m = jnp.min(dist, axis=1, keepdims=True)
        s = s + jnp.sum(m)
        ss = ss + jnp.sum(m * m)
        hit = dist <= m
        first = jnp.min(jnp.where(hit, col, t), axis=1, keepdims=True)
        dist = jnp.where(col == first, jnp.float32(jnp.inf), dist)

    lane = jax.lax.broadcasted_iota(jnp.int32, (1, 128), 1)
    vec = jnp.where(lane == 0, s, jnp.where(lane == 1, ss, 0.0))

    @pl.when(i == 0)
    def _init():
        out_ref[...] = vec

    @pl.when(i > 0)
    def _acc():
        out_ref[...] += vec


@jax.jit
def kernel(latent):
    if latent.shape[0] < 2:
        return jnp.asarray(0.0, dtype=latent.dtype)
    b, t, d = latent.shape
    sample = latent[0].astype(jnp.float32)
    k = min(_K, t - 1)
    n_blocks = t // _BLOCK_ROWS

    moments = pl.pallas_call(
        functools.partial(_knn_moments_kernel, t=t, k=k, n_blocks=n_blocks),
        grid=(n_blocks,),
        in_specs=[
            pl.BlockSpec((_BLOCK_ROWS, d), lambda i: (i, 0)),
            pl.BlockSpec((t, d), lambda i: (0, 0)),
        ],
        out_specs=pl.BlockSpec((1, 128), lambda i: (0, 0)),
        out_shape=jax.ShapeDtypeStruct((1, 128), jnp.float32),
    )(sample, sample)

    n = jnp.float32(t * k)
    s = moments[0, 0]
    ss = moments[0, 1]
    var = (ss - s * s / n) / (n - 1.0)
    return (-jnp.sqrt(jnp.maximum(var, 0.0))).astype(latent.dtype)


# speedup vs baseline: 13.7550x; 1.2921x over previous
import functools

import jax
import jax.numpy as jnp
from jax.experimental import pallas as pl
from jax.experimental.pallas import tpu as pltpu

_BLOCK_ROWS = 512
_K = 5


def _knn_moments_kernel(rows_ref, full_ref, out_ref, *, t, k, n_blocks):
    i = pl.program_id(0)

    x = rows_ref[...]
    y = full_ref[...]
    sqx = jnp.sum(x * x, axis=1, keepdims=True)
    sqy = jnp.sum(y * y, axis=1, keepdims=True).T
    xy = jax.lax.dot_general(
        x.astype(jnp.bfloat16), y.astype(jnp.bfloat16),
        dimension_numbers=(((1,), (1,)), ((), ())),
        preferred_element_type=jnp.float32,
    )
    d2 = jnp.maximum(sqx + sqy - 2.0 * xy, 0.0)

    col = jax.lax.broadcasted_iota(jnp.int32, d2.shape, 1)
    s = jnp.float32(0.0)
    ss = jnp.float32(0.0)
    for j in range(k):
        m2 = jnp.min(d2, axis=1, keepdims=True)
        m = jnp.where(m2 > 0, jnp.sqrt(jnp.where(m2 > 0, m2, 1.0)), 0.0)
        s = s + jnp.sum(m)
        ss = ss + jnp.sum(m * m)
        if j + 1 < k:
            hit = d2 <= m2
            first = jnp.min(jnp.where(hit, col, t), axis=1, keepdims=True)
            d2 = jnp.where(col == first, jnp.float32(jnp.inf), d2)

    lane = jax.lax.broadcasted_iota(jnp.int32, (1, 128), 1)
    vec = jnp.where(lane == 0, s, jnp.where(lane == 1, ss, 0.0))

    @pl.when(i == 0)
    def _init():
        out_ref[...] = vec

    @pl.when(i > 0)
    def _acc():
        out_ref[...] += vec


@jax.jit
def kernel(latent):
    if latent.shape[0] < 2:
        return jnp.asarray(0.0, dtype=latent.dtype)
    b, t, d = latent.shape
    sample = latent[0].astype(jnp.float32)
    k = min(_K, t - 1)
    n_blocks = t // _BLOCK_ROWS

    moments = pl.pallas_call(
        functools.partial(_knn_moments_kernel, t=t, k=k, n_blocks=n_blocks),
        grid=(n_blocks,),
        in_specs=[
            pl.BlockSpec((_BLOCK_ROWS, d), lambda i: (i, 0)),
            pl.BlockSpec((t, d), lambda i: (0, 0)),
        ],
        out_specs=pl.BlockSpec((1, 128), lambda i: (0, 0)),
        out_shape=jax.ShapeDtypeStruct((1, 128), jnp.float32),
    )(sample, sample)

    n = jnp.float32(t * k)
    s = moments[0, 0]
    ss = moments[0, 1]
    var = (ss - s * s / n) / (n - 1.0)
    return (-jnp.sqrt(jnp.maximum(var, 0.0))).astype(latent.dtype)


# speedup vs baseline: 16.5046x; 1.1999x over previous
import functools

import jax
import jax.numpy as jnp
from jax.experimental import pallas as pl
from jax.experimental.pallas import tpu as pltpu

_BLOCK_ROWS = 512
_K = 5


def _knn_moments_kernel(rows_ref, full_ref, out_ref, *, t, k, n_blocks):
    i = pl.program_id(0)

    x = rows_ref[...]
    y = full_ref[...]
    sqx = jnp.sum(x * x, axis=1, keepdims=True)
    sqy = jnp.sum(y * y, axis=1, keepdims=True).T
    xy = jax.lax.dot_general(
        x.astype(jnp.bfloat16), y.astype(jnp.bfloat16),
        dimension_numbers=(((1,), (1,)), ((), ())),
        preferred_element_type=jnp.float32,
    )
    r = xy.shape[0]
    chunk = 128
    n_chunks = t // chunk
    state = [jnp.full((r, chunk), jnp.inf, jnp.float32) for _ in range(k)]
    for c in range(n_chunks):
        sl = slice(c * chunk, (c + 1) * chunk)
        v = jnp.maximum(sqx + sqy[:, sl] - 2.0 * xy[:, sl], 0.0)
        for j in range(k):
            lo = jnp.minimum(state[j], v)
            v = jnp.maximum(state[j], v)
            state[j] = lo

    cand = jnp.concatenate(state, axis=1)
    width = k * chunk
    col = jax.lax.broadcasted_iota(jnp.int32, cand.shape, 1)
    s = jnp.float32(0.0)
    ss = jnp.float32(0.0)
    for j in range(k):
        m2 = jnp.min(cand, axis=1, keepdims=True)
        m = jnp.where(m2 > 0, jnp.sqrt(jnp.where(m2 > 0, m2, 1.0)), 0.0)
        s = s + jnp.sum(m)
        ss = ss + jnp.sum(m * m)
        if j + 1 < k:
            hit = cand <= m2
            first = jnp.min(jnp.where(hit, col, width), axis=1, keepdims=True)
            cand = jnp.where(col == first, jnp.float32(jnp.inf), cand)

    lane = jax.lax.broadcasted_iota(jnp.int32, (1, 128), 1)
    vec = jnp.where(lane == 0, s, jnp.where(lane == 1, ss, 0.0))

    @pl.when(i == 0)
    def _init():
        out_ref[...] = vec

    @pl.when(i > 0)
    def _acc():
        out_ref[...] += vec


@jax.jit
def kernel(latent):
    if latent.shape[0] < 2:
        return jnp.asarray(0.0, dtype=latent.dtype)
    b, t, d = latent.shape
    sample = latent[0].astype(jnp.float32)
    k = min(_K, t - 1)
    n_blocks = t // _BLOCK_ROWS

    moments = pl.pallas_call(
        functools.partial(_knn_moments_kernel, t=t, k=k, n_blocks=n_blocks),
        grid=(n_blocks,),
        in_specs=[
            pl.BlockSpec((_BLOCK_ROWS, d), lambda i: (i, 0)),
            pl.BlockSpec((t, d), lambda i: (0, 0)),
        ],
        out_specs=pl.BlockSpec((1, 128), lambda i: (0, 0)),
        out_shape=jax.ShapeDtypeStruct((1, 128), jnp.float32),
    )(sample, sample)

    n = jnp.float32(t * k)
    s = moments[0, 0]
    ss = moments[0, 1]
    var = (ss - s * s / n) / (n - 1.0)
    return (-jnp.sqrt(jnp.maximum(var, 0.0))).astype(latent.dtype)


# speedup vs baseline: 18.3529x; 1.1120x over previous
import functools

import jax
import jax.numpy as jnp
from jax.experimental import pallas as pl
from jax.experimental.pallas import tpu as pltpu

_BLOCK_ROWS = 512
_K = 5


def _knn_moments_kernel(rows_ref, full_ref, out_ref, *, t, k, n_blocks):
    i = pl.program_id(0)

    x = rows_ref[...]
    y = full_ref[...]
    sqx = jnp.sum(x * x, axis=1, keepdims=True)
    sqy = jnp.sum(y * y, axis=1, keepdims=True).T
    xy = jax.lax.dot_general(
        (-2.0 * x).astype(jnp.bfloat16), y.astype(jnp.bfloat16),
        dimension_numbers=(((1,), (1,)), ((), ())),
        preferred_element_type=jnp.float32,
    )
    r = xy.shape[0]
    chunk = 128
    sub = 64
    n_chunks = t // chunk
    s = jnp.float32(0.0)
    ss = jnp.float32(0.0)
    width = k * chunk
    col = jax.lax.broadcasted_iota(jnp.int32, (sub, width), 1)
    for st in range(r // sub):
        rows = slice(st * sub, (st + 1) * sub)
        sqx_r = sqx[rows]
        state = [jnp.full((sub, chunk), jnp.inf, jnp.float32)
                 for _ in range(k)]
        for c in range(n_chunks):
            sl = slice(c * chunk, (c + 1) * chunk)
            v = jnp.maximum((sqx_r + sqy[:, sl]) + xy[rows, sl], 0.0)
            for j in range(k):
                lo = jnp.minimum(state[j], v)
                v = jnp.maximum(state[j], v)
                state[j] = lo

        cand = jnp.concatenate(state, axis=1)
        for j in range(k):
            m2 = jnp.min(cand, axis=1, keepdims=True)
            m = jnp.where(m2 > 0, jnp.sqrt(jnp.where(m2 > 0, m2, 1.0)), 0.0)
            s = s + jnp.sum(m)
            ss = ss + jnp.sum(m * m)
            if j + 1 < k:
                hit = cand <= m2
                first = jnp.min(jnp.where(hit, col, width), axis=1,
                                keepdims=True)
                cand = jnp.where(col == first, jnp.float32(jnp.inf), cand)

    lane = jax.lax.broadcasted_iota(jnp.int32, (1, 128), 1)
    vec = jnp.where(lane == 0, s, jnp.where(lane == 1, ss, 0.0))

    @pl.when(i == 0)
    def _init():
        out_ref[...] = vec

    @pl.when(i > 0)
    def _acc():
        out_ref[...] += vec


@jax.jit
def kernel(latent):
    if latent.shape[0] < 2:
        return jnp.asarray(0.0, dtype=latent.dtype)
    b, t, d = latent.shape
    sample = latent[0].astype(jnp.float32)
    k = min(_K, t - 1)
    n_blocks = t // _BLOCK_ROWS

    moments = pl.pallas_call(
        functools.partial(_knn_moments_kernel, t=t, k=k, n_blocks=n_blocks),
        grid=(n_blocks,),
        in_specs=[
            pl.BlockSpec((_BLOCK_ROWS, d), lambda i: (i, 0)),
            pl.BlockSpec((t, d), lambda i: (0, 0)),
        ],
        out_specs=pl.BlockSpec((1, 128), lambda i: (0, 0)),
        out_shape=jax.ShapeDtypeStruct((1, 128), jnp.float32),
    )(sample, sample)

    n = jnp.float32(t * k)
    s = moments[0, 0]
    ss = moments[0, 1]
    var = (ss - s * s / n) / (n - 1.0)
    return (-jnp.sqrt(jnp.maximum(var, 0.0))).astype(latent.dtype)
